# baseline (speedup 1.0000x reference)
import functools

import jax
import jax.numpy as jnp
from jax import lax
from jax.experimental import pallas as pl
from jax.experimental.pallas import tpu as pltpu
from jax.experimental.pallas import tpu_sc as plsc

T = 8192
D = 1024
E = 8
FF = 4096
KTOP = 2
A = T * KTOP
SQ = 128
BT = 256
L = A + E * BT
NL = L // BT
FC = 2048

NC = 2
NS = 16
NW = NC * NS
RPW = L // NW
GC = 32
TPW = T // NW
CT = 32


def _router_body(x_ref, wr_ref, logits_ref, i12_ref, w12_ref):
    x = x_ref[...]
    wr = wr_ref[...]
    logits = jax.lax.dot_general(
        x, wr, (((1,), (1,)), ((), ())), preferred_element_type=jnp.float32)
    logits_ref[...] = logits
    m = jnp.max(logits, axis=1, keepdims=True)
    unnorm = jnp.exp(logits - m)
    p = unnorm / jnp.sum(unnorm, axis=1, keepdims=True)
    cols = jax.lax.broadcasted_iota(jnp.int32, p.shape, 1)
    i1 = jnp.argmax(p, axis=1)[:, None]
    p_m = jnp.where(cols == i1, -jnp.inf, p)
    i2 = jnp.argmax(p_m, axis=1)[:, None]
    w1 = jnp.sum(jnp.where(cols == i1, p, 0.0), axis=1, keepdims=True)
    w2 = jnp.sum(jnp.where(cols == i2, p, 0.0), axis=1, keepdims=True)
    i12_ref[...] = jnp.concatenate([i1, i2], axis=1)
    w12_ref[...] = jnp.concatenate([w1, w2], axis=1)


def _dispatch_body(ti_ref, dest_ref, eid_ref):
    ti = ti_ref[...]
    r_i = jax.lax.broadcasted_iota(jnp.int32, (SQ, SQ), 0)
    c_i = jax.lax.broadcasted_iota(jnp.int32, (SQ, SQ), 1)
    incl = (r_i <= c_i).astype(jnp.float32)
    strict = (c_i < r_i).astype(jnp.float32)

    masks, counts = [], []
    ranks = []
    for e in range(E):
        mf = (ti == e).astype(jnp.float32)
        cum = jax.lax.dot_general(
            mf, incl, (((1,), (0,)), ((), ())),
            preferred_element_type=jnp.float32)
        rowsum = cum[:, SQ - 1:SQ]
        rowpref = jax.lax.dot_general(
            strict, rowsum, (((1,), (0,)), ((), ())),
            preferred_element_type=jnp.float32)
        rank = cum - mf + rowpref
        masks.append(mf)
        ranks.append(rank)
        counts.append(jnp.sum(mf))

    base = 0.0
    dest = jnp.zeros((SQ, SQ), jnp.float32)
    bases = []
    for e in range(E):
        bases.append(base)
        dest = dest + masks[e] * (ranks[e] + base)
        padded = jnp.ceil(counts[e] / BT) * BT
        base = base + padded
    dest_ref[...] = dest.astype(jnp.int32)

    ts = jax.lax.broadcasted_iota(jnp.int32, (1, SQ), 1).astype(
        jnp.float32) * BT
    eid = jnp.zeros((1, SQ), jnp.int32)
    for e in range(1, E):
        eid = eid + (ts >= bases[e]).astype(jnp.int32)
    eid_ref[...] = eid


def _sc_dispatch_body(dest_hbm, tw_hbm, x_hbm, xs_hbm, ws_hbm,
                      zt_v, zw_v, dst8_v, tw_v, vals_v,
                      idxall_v, rows_a, rows_b, rows_c, wslice_v,
                      stok_sh, sw_sh, sem, sem2):
    cid = lax.axis_index("c")
    sid = lax.axis_index("s")

    zs = L // NS
    sqr = SQ // NS
    ac = sqr * SQ

    def zb(i, c):
        zt_v[pl.ds(i * 16, 16)] = jnp.zeros((16,), jnp.int32)
        zw_v[pl.ds(i * 16, 16)] = jnp.zeros((16,), jnp.float32)
        return c
    lax.fori_loop(0, zs // 16, zb, 0)
    pltpu.sync_copy(zt_v, stok_sh.at[pl.ds(sid * zs, zs)])
    pltpu.sync_copy(zw_v, sw_sh.at[pl.ds(sid * zs, zs)])

    pltpu.sync_copy(dest_hbm.at[pl.ds(sid * sqr, sqr)], dst8_v)
    pltpu.sync_copy(tw_hbm.at[pl.ds(sid * ac, ac)], tw_v)

    def vb(i, c):
        va = sid * ac + i * 16 + jax.lax.iota(jnp.int32, 16)
        vals_v[pl.ds(i * 16, 16)] = jnp.bitwise_and(va, T - 1)
        return c
    lax.fori_loop(0, ac // 16, vb, 0)

    plsc.subcore_barrier()

    hs = []
    for j in range(sqr):
        hs.append(pltpu.async_copy(
            vals_v.at[pl.ds(j * SQ, SQ)], stok_sh.at[dst8_v.at[j]], sem))
        hs.append(pltpu.async_copy(
            tw_v.at[pl.ds(j * SQ, SQ)], sw_sh.at[dst8_v.at[j]], sem))
    for h in hs:
        h.wait()

    plsc.subcore_barrier()

    wid = cid * NS + sid
    base = wid * RPW
    nch = RPW // GC
    rowbufs = (rows_a, rows_b, rows_c)
    nb = len(rowbufs)

    pltpu.sync_copy(stok_sh.at[pl.ds(base, RPW)], idxall_v)
    pltpu.sync_copy(sw_sh.at[pl.ds(base, RPW)], wslice_v)
    wsh = pltpu.async_copy(wslice_v, ws_hbm.at[pl.ds(base, RPW)], sem2)

    def start(c):
        return pltpu.async_copy(
            x_hbm.at[idxall_v.at[pl.ds(c * GC, GC)]], rowbufs[c % nb], sem)

    pend_g = [None] * nb
    pend_w = [None] * nb
    pend_g[0] = start(0)
    if nch > 1:
        pend_g[1] = start(1)
    for c in range(nch):
        b = c % nb
        nxt_c = c + 2
        if nxt_c < nch:
            b2 = nxt_c % nb
            if pend_w[b2] is not None:
                pend_w[b2].wait()
            pend_g[b2] = start(nxt_c)
        pend_g[b].wait()
        pend_w[b] = pltpu.async_copy(
            rowbufs[b], xs_hbm.at[pl.ds(base + c * GC, GC)], sem2)
    for h in pend_w:
        if h is not None:
            h.wait()
    wsh.wait()


def _moe_ffn_body(eid_ref, xs_ref, ws_ref, g_ref, u_ref, d_ref, ys_ref):
    xb = xs_ref[...].astype(jnp.bfloat16)
    acc = jnp.zeros((BT, D), jnp.float32)
    for f in range(FF // FC):
        gw = g_ref[0, pl.ds(f * FC, FC), :]
        uw = u_ref[0, pl.ds(f * FC, FC), :]
        g = jax.lax.dot_general(
            xb, gw, (((1,), (1,)), ((), ())),
            preferred_element_type=jnp.float32)
        u = jax.lax.dot_general(
            xb, uw, (((1,), (1,)), ((), ())),
            preferred_element_type=jnp.float32)
        h = ((g * jax.nn.sigmoid(g)) * u).astype(jnp.bfloat16)
        dw = d_ref[0, :, pl.ds(f * FC, FC)]
        acc = acc + jax.lax.dot_general(
            h, dw, (((1,), (1,)), ((), ())),
            preferred_element_type=jnp.float32)
    ys_ref[...] = ws_ref[...] * acc


def _sc_combine_body(ys_hbm, dest_hbm, out_hbm,
                     idxa_v, idxb_v, bufa_v, bufb_v, sem):
    cid = lax.axis_index("c")
    sid = lax.axis_index("s")
    wid = cid * NS + sid

    def chunk_body(c, carry):
        t0 = wid * TPW + c * CT
        pltpu.sync_copy(dest_hbm.at[pl.ds(t0, CT)], idxa_v)
        pltpu.sync_copy(dest_hbm.at[pl.ds(T + t0, CT)], idxb_v)
        pltpu.async_copy(ys_hbm.at[idxa_v], bufa_v, sem).wait()
        pltpu.async_copy(ys_hbm.at[idxb_v], bufb_v, sem).wait()

        def add_row(r, cc):
            for k4 in range(D // 64):
                for j in range(4):
                    sl = pl.ds(k4 * 64 + j * 16, 16)
                    bufa_v[r, sl] = bufa_v[r, sl] + bufb_v[r, sl]
            return cc
        lax.fori_loop(0, CT, add_row, 0)
        pltpu.sync_copy(bufa_v, out_hbm.at[pl.ds(t0, CT)])
        return carry
    lax.fori_loop(0, TPW // CT, chunk_body, 0)


def kernel(x, Wr, gate, up, down):
    b, s, d = x.shape
    xf = x.reshape(T, D)

    bt_r = 1024
    logits, i12, w12 = pl.pallas_call(
        _router_body,
        grid=(T // bt_r,),
        in_specs=[
            pl.BlockSpec((bt_r, D), lambda i: (i, 0)),
            pl.BlockSpec((E, D), lambda i: (0, 0)),
        ],
        out_specs=[
            pl.BlockSpec((bt_r, E), lambda i: (i, 0)),
            pl.BlockSpec((bt_r, KTOP), lambda i: (i, 0)),
            pl.BlockSpec((bt_r, KTOP), lambda i: (i, 0)),
        ],
        out_shape=[
            jax.ShapeDtypeStruct((T, E), jnp.float32),
            jax.ShapeDtypeStruct((T, KTOP), jnp.int32),
            jax.ShapeDtypeStruct((T, KTOP), jnp.float32),
        ],
    )(xf, Wr)

    ti_sq = i12.T.reshape(SQ, SQ)
    tw_flat = w12.T.reshape(A)

    dest_sq, eid_pad = pl.pallas_call(
        _dispatch_body,
        grid=(1,),
        in_specs=[pl.BlockSpec((SQ, SQ), lambda i: (0, 0))],
        out_specs=[
            pl.BlockSpec((SQ, SQ), lambda i: (0, 0)),
            pl.BlockSpec((1, SQ), lambda i: (0, 0)),
        ],
        out_shape=[
            jax.ShapeDtypeStruct((SQ, SQ), jnp.int32),
            jax.ShapeDtypeStruct((1, SQ), jnp.int32),
        ],
    )(ti_sq)
    dest_flat = dest_sq.reshape(A)
    tile_eid = eid_pad.reshape(SQ)[:NL]

    sc_mesh = plsc.VectorSubcoreMesh(core_axis_name="c", subcore_axis_name="s", num_cores=NC, num_subcores=NS)
    xs, ws = pl.kernel(
        _sc_dispatch_body,
        out_type=[
            jax.ShapeDtypeStruct((L, D), jnp.float32),
            jax.ShapeDtypeStruct((L,), jnp.float32),
        ],
        mesh=sc_mesh,
        scratch_types=[
            pltpu.VMEM((L // NS,), jnp.int32),
            pltpu.VMEM((L // NS,), jnp.float32),
            pltpu.VMEM((SQ // NS, SQ), jnp.int32),
            pltpu.VMEM((SQ // NS * SQ,), jnp.float32),
            pltpu.VMEM((SQ // NS * SQ,), jnp.int32),
            pltpu.VMEM((RPW,), jnp.int32),
            pltpu.VMEM((GC, D), jnp.float32),
            pltpu.VMEM((GC, D), jnp.float32),
            pltpu.VMEM((GC, D), jnp.float32),
            pltpu.VMEM((RPW,), jnp.float32),
            pltpu.VMEM_SHARED((L,), jnp.int32),
            pltpu.VMEM_SHARED((L,), jnp.float32),
            pltpu.SemaphoreType.DMA,
            pltpu.SemaphoreType.DMA,
        ],
    )(dest_sq, tw_flat, xf)

    gate_b = gate.astype(jnp.bfloat16)
    up_b = up.astype(jnp.bfloat16)
    down_b = down.astype(jnp.bfloat16)
    ws2 = ws.reshape(L, 1)

    grid_spec = pltpu.PrefetchScalarGridSpec(
        num_scalar_prefetch=1,
        grid=(NL,),
        in_specs=[
            pl.BlockSpec((BT, D), lambda i, eid: (i, 0)),
            pl.BlockSpec((BT, 1), lambda i, eid: (i, 0)),
            pl.BlockSpec((1, FF, D), lambda i, eid: (eid[i], 0, 0)),
            pl.BlockSpec((1, FF, D), lambda i, eid: (eid[i], 0, 0)),
            pl.BlockSpec((1, D, FF), lambda i, eid: (eid[i], 0, 0)),
        ],
        out_specs=pl.BlockSpec((BT, D), lambda i, eid: (i, 0)),
    )
    ys = pl.pallas_call(
        _moe_ffn_body,
        grid_spec=grid_spec,
        out_shape=jax.ShapeDtypeStruct((L, D), jnp.float32),
        compiler_params=pltpu.CompilerParams(
            dimension_semantics=("arbitrary",)),
    )(tile_eid, xs, ws2, gate_b, up_b, down_b)

    final = pl.kernel(
        _sc_combine_body,
        out_type=jax.ShapeDtypeStruct((T, D), jnp.float32),
        mesh=sc_mesh,
        scratch_types=[
            pltpu.VMEM((CT,), jnp.int32),
            pltpu.VMEM((CT,), jnp.int32),
            pltpu.VMEM((CT, D), jnp.float32),
            pltpu.VMEM((CT, D), jnp.float32),
            pltpu.SemaphoreType.DMA,
        ],
    )(ys, dest_flat)

    return final.reshape(b, s, d), logits


# speedup vs baseline: 1.0329x; 1.0329x over previous
import functools

import jax
import jax.numpy as jnp
from jax import lax
from jax.experimental import pallas as pl
from jax.experimental.pallas import tpu as pltpu
from jax.experimental.pallas import tpu_sc as plsc

T = 8192
D = 1024
E = 8
FF = 4096
KTOP = 2
A = T * KTOP
SQ = 128
BT = 256
L = A + E * BT
NL = L // BT
FC = 2048

NC = 2
NS = 16
NW = NC * NS
RPW = L // NW
GC = 32
TPW = T // NW
CT = 16


def _router_body(x_ref, wr_ref, logits_ref, i12_ref, w12_ref):
    x = x_ref[...]
    wr = wr_ref[...]
    logits = jax.lax.dot_general(
        x, wr, (((1,), (1,)), ((), ())), preferred_element_type=jnp.float32)
    logits_ref[...] = logits
    m = jnp.max(logits, axis=1, keepdims=True)
    unnorm = jnp.exp(logits - m)
    p = unnorm / jnp.sum(unnorm, axis=1, keepdims=True)
    cols = jax.lax.broadcasted_iota(jnp.int32, p.shape, 1)
    i1 = jnp.argmax(p, axis=1)[:, None]
    p_m = jnp.where(cols == i1, -jnp.inf, p)
    i2 = jnp.argmax(p_m, axis=1)[:, None]
    w1 = jnp.sum(jnp.where(cols == i1, p, 0.0), axis=1, keepdims=True)
    w2 = jnp.sum(jnp.where(cols == i2, p, 0.0), axis=1, keepdims=True)
    i12_ref[...] = jnp.concatenate([i1, i2], axis=1)
    w12_ref[...] = jnp.concatenate([w1, w2], axis=1)


def _dispatch_body(ti_ref, dest_ref, eid_ref):
    ti = ti_ref[...]
    r_i = jax.lax.broadcasted_iota(jnp.int32, (SQ, SQ), 0)
    c_i = jax.lax.broadcasted_iota(jnp.int32, (SQ, SQ), 1)
    incl = (r_i <= c_i).astype(jnp.float32)
    strict = (c_i < r_i).astype(jnp.float32)

    masks, counts = [], []
    ranks = []
    for e in range(E):
        mf = (ti == e).astype(jnp.float32)
        cum = jax.lax.dot_general(
            mf, incl, (((1,), (0,)), ((), ())),
            preferred_element_type=jnp.float32)
        rowsum = cum[:, SQ - 1:SQ]
        rowpref = jax.lax.dot_general(
            strict, rowsum, (((1,), (0,)), ((), ())),
            preferred_element_type=jnp.float32)
        rank = cum - mf + rowpref
        masks.append(mf)
        ranks.append(rank)
        counts.append(jnp.sum(mf))

    base = 0.0
    dest = jnp.zeros((SQ, SQ), jnp.float32)
    bases = []
    for e in range(E):
        bases.append(base)
        dest = dest + masks[e] * (ranks[e] + base)
        padded = jnp.ceil(counts[e] / BT) * BT
        base = base + padded
    dest_ref[...] = dest.astype(jnp.int32)

    ts = jax.lax.broadcasted_iota(jnp.int32, (1, SQ), 1).astype(
        jnp.float32) * BT
    eid = jnp.zeros((1, SQ), jnp.int32)
    for e in range(1, E):
        eid = eid + (ts >= bases[e]).astype(jnp.int32)
    eid_ref[...] = eid


def _sc_dispatch_body(dest_hbm, tw_hbm, x_hbm, xs_hbm, ws_hbm,
                      zt_v, zw_v, dst8_v, tw_v, vals_v,
                      idxall_v, rows_a, rows_b, rows_c, wslice_v,
                      stok_sh, sw_sh, sem, sem2):
    cid = lax.axis_index("c")
    sid = lax.axis_index("s")

    zs = L // NS
    sqr = SQ // NS
    ac = sqr * SQ

    def zb(i, c):
        zt_v[pl.ds(i * 16, 16)] = jnp.zeros((16,), jnp.int32)
        zw_v[pl.ds(i * 16, 16)] = jnp.zeros((16,), jnp.float32)
        return c
    lax.fori_loop(0, zs // 16, zb, 0)
    pltpu.sync_copy(zt_v, stok_sh.at[pl.ds(sid * zs, zs)])
    pltpu.sync_copy(zw_v, sw_sh.at[pl.ds(sid * zs, zs)])

    pltpu.sync_copy(dest_hbm.at[pl.ds(sid * sqr, sqr)], dst8_v)
    pltpu.sync_copy(tw_hbm.at[pl.ds(sid * ac, ac)], tw_v)

    def vb(i, c):
        va = sid * ac + i * 16 + jax.lax.iota(jnp.int32, 16)
        vals_v[pl.ds(i * 16, 16)] = jnp.bitwise_and(va, T - 1)
        return c
    lax.fori_loop(0, ac // 16, vb, 0)

    plsc.subcore_barrier()

    hs = []
    for j in range(sqr):
        hs.append(pltpu.async_copy(
            vals_v.at[pl.ds(j * SQ, SQ)], stok_sh.at[dst8_v.at[j]], sem))
        hs.append(pltpu.async_copy(
            tw_v.at[pl.ds(j * SQ, SQ)], sw_sh.at[dst8_v.at[j]], sem))
    for h in hs:
        h.wait()

    plsc.subcore_barrier()

    wid = cid * NS + sid
    base = wid * RPW
    nch = RPW // GC
    rowbufs = (rows_a, rows_b, rows_c)
    nb = len(rowbufs)

    pltpu.sync_copy(stok_sh.at[pl.ds(base, RPW)], idxall_v)
    pltpu.sync_copy(sw_sh.at[pl.ds(base, RPW)], wslice_v)
    wsh = pltpu.async_copy(wslice_v, ws_hbm.at[pl.ds(base, RPW)], sem2)

    def start(c):
        return pltpu.async_copy(
            x_hbm.at[idxall_v.at[pl.ds(c * GC, GC)]], rowbufs[c % nb], sem)

    pend_g = [None] * nb
    pend_w = [None] * nb
    pend_g[0] = start(0)
    if nch > 1:
        pend_g[1] = start(1)
    for c in range(nch):
        b = c % nb
        nxt_c = c + 2
        if nxt_c < nch:
            b2 = nxt_c % nb
            if pend_w[b2] is not None:
                pend_w[b2].wait()
            pend_g[b2] = start(nxt_c)
        pend_g[b].wait()
        pend_w[b] = pltpu.async_copy(
            rowbufs[b], xs_hbm.at[pl.ds(base + c * GC, GC)], sem2)
    for h in pend_w:
        if h is not None:
            h.wait()
    wsh.wait()


def _moe_ffn_body(eid_ref, xs_ref, ws_ref, g_ref, u_ref, d_ref, ys_ref):
    xb = xs_ref[...].astype(jnp.bfloat16)
    acc = jnp.zeros((BT, D), jnp.float32)
    for f in range(FF // FC):
        gw = g_ref[0, pl.ds(f * FC, FC), :]
        uw = u_ref[0, pl.ds(f * FC, FC), :]
        g = jax.lax.dot_general(
            xb, gw, (((1,), (1,)), ((), ())),
            preferred_element_type=jnp.float32)
        u = jax.lax.dot_general(
            xb, uw, (((1,), (1,)), ((), ())),
            preferred_element_type=jnp.float32)
        h = ((g * jax.nn.sigmoid(g)) * u).astype(jnp.bfloat16)
        dw = d_ref[0, :, pl.ds(f * FC, FC)]
        acc = acc + jax.lax.dot_general(
            h, dw, (((1,), (1,)), ((), ())),
            preferred_element_type=jnp.float32)
    ys_ref[...] = ws_ref[...] * acc


def _sc_combine_body(ys_hbm, dest_hbm, out_hbm,
                     idxa_v, idxb_v, bufa0, bufb0, bufa1, bufb1,
                     sem, sem2):
    cid = lax.axis_index("c")
    sid = lax.axis_index("s")
    wid = cid * NS + sid
    t0w = wid * TPW
    nch = TPW // CT
    bufa = (bufa0, bufa1)
    bufb = (bufb0, bufb1)

    pltpu.sync_copy(dest_hbm.at[pl.ds(t0w, TPW)], idxa_v)
    pltpu.sync_copy(dest_hbm.at[pl.ds(T + t0w, TPW)], idxb_v)

    def startc(c):
        b = c % 2
        ha = pltpu.async_copy(
            ys_hbm.at[idxa_v.at[pl.ds(c * CT, CT)]], bufa[b], sem)
        hb = pltpu.async_copy(
            ys_hbm.at[idxb_v.at[pl.ds(c * CT, CT)]], bufb[b], sem)
        return ha, hb

    g = [startc(0), startc(1) if nch > 1 else None]
    w = [None, None]
    for c in range(nch):
        b = c % 2
        ga, gb = g[b]
        ga.wait()
        gb.wait()

        def add_row(r, cc):
            for k4 in range(D // 64):
                for j in range(4):
                    sl = pl.ds(k4 * 64 + j * 16, 16)
                    bufa[b][r, sl] = bufa[b][r, sl] + bufb[b][r, sl]
            return cc
        lax.fori_loop(0, CT, add_row, 0)
        w[b] = pltpu.async_copy(
            bufa[b], out_hbm.at[pl.ds(t0w + c * CT, CT)], sem2)
        if c + 2 < nch:
            w[b].wait()
            g[b] = startc(c + 2)
    for h in w:
        if h is not None:
            h.wait()


def kernel(x, Wr, gate, up, down):
    b, s, d = x.shape
    xf = x.reshape(T, D)

    bt_r = 1024
    logits, i12, w12 = pl.pallas_call(
        _router_body,
        grid=(T // bt_r,),
        in_specs=[
            pl.BlockSpec((bt_r, D), lambda i: (i, 0)),
            pl.BlockSpec((E, D), lambda i: (0, 0)),
        ],
        out_specs=[
            pl.BlockSpec((bt_r, E), lambda i: (i, 0)),
            pl.BlockSpec((bt_r, KTOP), lambda i: (i, 0)),
            pl.BlockSpec((bt_r, KTOP), lambda i: (i, 0)),
        ],
        out_shape=[
            jax.ShapeDtypeStruct((T, E), jnp.float32),
            jax.ShapeDtypeStruct((T, KTOP), jnp.int32),
            jax.ShapeDtypeStruct((T, KTOP), jnp.float32),
        ],
    )(xf, Wr)

    ti_sq = i12.T.reshape(SQ, SQ)
    tw_flat = w12.T.reshape(A)

    dest_sq, eid_pad = pl.pallas_call(
        _dispatch_body,
        grid=(1,),
        in_specs=[pl.BlockSpec((SQ, SQ), lambda i: (0, 0))],
        out_specs=[
            pl.BlockSpec((SQ, SQ), lambda i: (0, 0)),
            pl.BlockSpec((1, SQ), lambda i: (0, 0)),
        ],
        out_shape=[
            jax.ShapeDtypeStruct((SQ, SQ), jnp.int32),
            jax.ShapeDtypeStruct((1, SQ), jnp.int32),
        ],
    )(ti_sq)
    dest_flat = dest_sq.reshape(A)
    tile_eid = eid_pad.reshape(SQ)[:NL]

    sc_mesh = plsc.VectorSubcoreMesh(core_axis_name="c", subcore_axis_name="s", num_cores=NC, num_subcores=NS)
    xs, ws = pl.kernel(
        _sc_dispatch_body,
        out_type=[
            jax.ShapeDtypeStruct((L, D), jnp.float32),
            jax.ShapeDtypeStruct((L,), jnp.float32),
        ],
        mesh=sc_mesh,
        scratch_types=[
            pltpu.VMEM((L // NS,), jnp.int32),
            pltpu.VMEM((L // NS,), jnp.float32),
            pltpu.VMEM((SQ // NS, SQ), jnp.int32),
            pltpu.VMEM((SQ // NS * SQ,), jnp.float32),
            pltpu.VMEM((SQ // NS * SQ,), jnp.int32),
            pltpu.VMEM((RPW,), jnp.int32),
            pltpu.VMEM((GC, D), jnp.float32),
            pltpu.VMEM((GC, D), jnp.float32),
            pltpu.VMEM((GC, D), jnp.float32),
            pltpu.VMEM((RPW,), jnp.float32),
            pltpu.VMEM_SHARED((L,), jnp.int32),
            pltpu.VMEM_SHARED((L,), jnp.float32),
            pltpu.SemaphoreType.DMA,
            pltpu.SemaphoreType.DMA,
        ],
    )(dest_sq, tw_flat, xf)

    gate_b = gate.astype(jnp.bfloat16)
    up_b = up.astype(jnp.bfloat16)
    down_b = down.astype(jnp.bfloat16)
    ws2 = ws.reshape(L, 1)

    grid_spec = pltpu.PrefetchScalarGridSpec(
        num_scalar_prefetch=1,
        grid=(NL,),
        in_specs=[
            pl.BlockSpec((BT, D), lambda i, eid: (i, 0)),
            pl.BlockSpec((BT, 1), lambda i, eid: (i, 0)),
            pl.BlockSpec((1, FF, D), lambda i, eid: (eid[i], 0, 0)),
            pl.BlockSpec((1, FF, D), lambda i, eid: (eid[i], 0, 0)),
            pl.BlockSpec((1, D, FF), lambda i, eid: (eid[i], 0, 0)),
        ],
        out_specs=pl.BlockSpec((BT, D), lambda i, eid: (i, 0)),
    )
    ys = pl.pallas_call(
        _moe_ffn_body,
        grid_spec=grid_spec,
        out_shape=jax.ShapeDtypeStruct((L, D), jnp.float32),
        compiler_params=pltpu.CompilerParams(
            dimension_semantics=("arbitrary",)),
    )(tile_eid, xs, ws2, gate_b, up_b, down_b)

    final = pl.kernel(
        _sc_combine_body,
        out_type=jax.ShapeDtypeStruct((T, D), jnp.float32),
        mesh=sc_mesh,
        scratch_types=[
            pltpu.VMEM((TPW,), jnp.int32),
            pltpu.VMEM((TPW,), jnp.int32),
            pltpu.VMEM((CT, D), jnp.float32),
            pltpu.VMEM((CT, D), jnp.float32),
            pltpu.VMEM((CT, D), jnp.float32),
            pltpu.VMEM((CT, D), jnp.float32),
            pltpu.SemaphoreType.DMA,
            pltpu.SemaphoreType.DMA,
        ],
    )(ys, dest_flat)

    return final.reshape(b, s, d), logits


# speedup vs baseline: 1.0580x; 1.0244x over previous
import jax
import jax.numpy as jnp
from jax import lax
from jax.experimental import pallas as pl
from jax.experimental.pallas import tpu as pltpu
from jax.experimental.pallas import tpu_sc as plsc

T = 8192
D = 1024
E = 8
FF = 4096
KTOP = 2
A = T * KTOP
SQ = 128
BT = 256
L = A + E * BT
NL = L // BT
FC = 2048

NC = 2
NS = 16
NW = NC * NS
RPW = L // NW
GC = 32
TPW = T // NW
CT = 16


def _router_body(x_ref, wr_ref, logits_ref, i12_ref, w12_ref):
    x = x_ref[...]
    wr = wr_ref[...]
    logits = jax.lax.dot_general(
        x, wr, (((1,), (1,)), ((), ())), preferred_element_type=jnp.float32)
    logits_ref[...] = logits
    m = jnp.max(logits, axis=1, keepdims=True)
    unnorm = jnp.exp(logits - m)
    p = unnorm / jnp.sum(unnorm, axis=1, keepdims=True)
    cols = jax.lax.broadcasted_iota(jnp.int32, p.shape, 1)
    i1 = jnp.argmax(p, axis=1)[:, None]
    p_m = jnp.where(cols == i1, -jnp.inf, p)
    i2 = jnp.argmax(p_m, axis=1)[:, None]
    w1 = jnp.sum(jnp.where(cols == i1, p, 0.0), axis=1, keepdims=True)
    w2 = jnp.sum(jnp.where(cols == i2, p, 0.0), axis=1, keepdims=True)
    i12_ref[...] = jnp.concatenate([i1, i2], axis=1)
    w12_ref[...] = jnp.concatenate([w1, w2], axis=1)


def _dispatch_body(ti_ref, dest_ref, eid_ref, nlive_ref):
    ti = ti_ref[...]
    r_i = jax.lax.broadcasted_iota(jnp.int32, (SQ, SQ), 0)
    c_i = jax.lax.broadcasted_iota(jnp.int32, (SQ, SQ), 1)
    incl = (r_i <= c_i).astype(jnp.float32)
    strict = (c_i < r_i).astype(jnp.float32)

    masks, counts = [], []
    ranks = []
    for e in range(E):
        mf = (ti == e).astype(jnp.float32)
        cum = jax.lax.dot_general(
            mf, incl, (((1,), (0,)), ((), ())),
            preferred_element_type=jnp.float32)
        rowsum = cum[:, SQ - 1:SQ]
        rowpref = jax.lax.dot_general(
            strict, rowsum, (((1,), (0,)), ((), ())),
            preferred_element_type=jnp.float32)
        rank = cum - mf + rowpref
        masks.append(mf)
        ranks.append(rank)
        counts.append(jnp.sum(mf))

    base = 0.0
    dest = jnp.zeros((SQ, SQ), jnp.float32)
    bases = []
    for e in range(E):
        bases.append(base)
        dest = dest + masks[e] * (ranks[e] + base)
        padded = jnp.ceil(counts[e] / BT) * BT
        base = base + padded
    dest_ref[...] = dest.astype(jnp.int32)

    ts = jax.lax.broadcasted_iota(jnp.int32, (1, SQ), 1).astype(
        jnp.float32) * BT
    eid = jnp.zeros((1, SQ), jnp.int32)
    for e in range(1, E):
        eid = eid + (ts >= bases[e]).astype(jnp.int32)
    eid_ref[...] = eid
    nlive_ref[...] = jnp.full((1, 1), base / BT, jnp.float32).astype(jnp.int32)


def _sc_dispatch_body(dest_hbm, tw_hbm, x_hbm, xs_hbm, ws_hbm,
                      zt_v, zw_v, dst8_v, tw_v, vals_v,
                      idxall_v, rows_a, rows_b, rows_c, wslice_v,
                      stok_sh, sw_sh, sem, sem2):
    cid = lax.axis_index("c")
    sid = lax.axis_index("s")

    zs = L // NS
    sqr = SQ // NS
    ac = sqr * SQ

    def zb(i, c):
        zt_v[pl.ds(i * 16, 16)] = jnp.zeros((16,), jnp.int32)
        zw_v[pl.ds(i * 16, 16)] = jnp.zeros((16,), jnp.float32)
        return c
    lax.fori_loop(0, zs // 16, zb, 0)
    pltpu.sync_copy(zt_v, stok_sh.at[pl.ds(sid * zs, zs)])
    pltpu.sync_copy(zw_v, sw_sh.at[pl.ds(sid * zs, zs)])

    pltpu.sync_copy(dest_hbm.at[pl.ds(sid * sqr, sqr)], dst8_v)
    pltpu.sync_copy(tw_hbm.at[pl.ds(sid * ac, ac)], tw_v)

    def vb(i, c):
        va = sid * ac + i * 16 + jax.lax.iota(jnp.int32, 16)
        vals_v[pl.ds(i * 16, 16)] = jnp.bitwise_and(va, T - 1)
        return c
    lax.fori_loop(0, ac // 16, vb, 0)

    plsc.subcore_barrier()

    hs = []
    for j in range(sqr):
        hs.append(pltpu.async_copy(
            vals_v.at[pl.ds(j * SQ, SQ)], stok_sh.at[dst8_v.at[j]], sem))
        hs.append(pltpu.async_copy(
            tw_v.at[pl.ds(j * SQ, SQ)], sw_sh.at[dst8_v.at[j]], sem))
    for h in hs:
        h.wait()

    plsc.subcore_barrier()

    wid = cid * NS + sid
    base = wid * RPW
    nch = RPW // GC
    rowbufs = (rows_a, rows_b, rows_c)
    nb = len(rowbufs)

    pltpu.sync_copy(stok_sh.at[pl.ds(base, RPW)], idxall_v)
    pltpu.sync_copy(sw_sh.at[pl.ds(base, RPW)], wslice_v)
    wsh = pltpu.async_copy(wslice_v, ws_hbm.at[pl.ds(base, RPW)], sem2)

    def start(c):
        return pltpu.async_copy(
            x_hbm.at[idxall_v.at[pl.ds(c * GC, GC)]], rowbufs[c % nb], sem)

    pend_g = [None] * nb
    pend_w = [None] * nb
    pend_g[0] = start(0)
    if nch > 1:
        pend_g[1] = start(1)
    for c in range(nch):
        b = c % nb
        nxt_c = c + 2
        if nxt_c < nch:
            b2 = nxt_c % nb
            if pend_w[b2] is not None:
                pend_w[b2].wait()
            pend_g[b2] = start(nxt_c)
        pend_g[b].wait()
        pend_w[b] = pltpu.async_copy(
            rowbufs[b], xs_hbm.at[pl.ds(base + c * GC, GC)], sem2)
    for h in pend_w:
        if h is not None:
            h.wait()
    wsh.wait()


def _moe_ffn_body(eid_ref, nlive_ref, xs_ref, ws_ref, g_ref, u_ref, d_ref,
                  ys_ref):
    @pl.when(pl.program_id(0) < nlive_ref[0])
    def _():
        _moe_ffn_tile(xs_ref, ws_ref, g_ref, u_ref, d_ref, ys_ref)


def _moe_ffn_tile(xs_ref, ws_ref, g_ref, u_ref, d_ref, ys_ref):
    xb = xs_ref[...].astype(jnp.bfloat16)
    acc = jnp.zeros((BT, D), jnp.float32)
    for f in range(FF // FC):
        gw = g_ref[0, pl.ds(f * FC, FC), :]
        uw = u_ref[0, pl.ds(f * FC, FC), :]
        g = jax.lax.dot_general(
            xb, gw, (((1,), (1,)), ((), ())),
            preferred_element_type=jnp.float32)
        u = jax.lax.dot_general(
            xb, uw, (((1,), (1,)), ((), ())),
            preferred_element_type=jnp.float32)
        h = ((g * jax.nn.sigmoid(g)) * u).astype(jnp.bfloat16)
        dw = d_ref[0, :, pl.ds(f * FC, FC)]
        acc = acc + jax.lax.dot_general(
            h, dw, (((1,), (1,)), ((), ())),
            preferred_element_type=jnp.float32)
    ys_ref[...] = ws_ref[...] * acc


def _sc_combine_body(ys_hbm, dest_hbm, out_hbm,
                     idxa_v, idxb_v, bufa0, bufb0, bufa1, bufb1,
                     sem, sem2):
    cid = lax.axis_index("c")
    sid = lax.axis_index("s")
    wid = cid * NS + sid
    t0w = wid * TPW
    nch = TPW // CT
    bufa = (bufa0, bufa1)
    bufb = (bufb0, bufb1)

    pltpu.sync_copy(dest_hbm.at[pl.ds(t0w, TPW)], idxa_v)
    pltpu.sync_copy(dest_hbm.at[pl.ds(T + t0w, TPW)], idxb_v)

    def startc(c):
        b = c % 2
        ha = pltpu.async_copy(
            ys_hbm.at[idxa_v.at[pl.ds(c * CT, CT)]], bufa[b], sem)
        hb = pltpu.async_copy(
            ys_hbm.at[idxb_v.at[pl.ds(c * CT, CT)]], bufb[b], sem)
        return ha, hb

    g = [startc(0), startc(1) if nch > 1 else None]
    w = [None, None]
    for c in range(nch):
        b = c % 2
        ga, gb = g[b]
        ga.wait()
        gb.wait()

        def add_row(r, cc):
            for k4 in range(D // 64):
                for j in range(4):
                    sl = pl.ds(k4 * 64 + j * 16, 16)
                    bufa[b][r, sl] = bufa[b][r, sl] + bufb[b][r, sl]
            return cc
        lax.fori_loop(0, CT, add_row, 0)
        w[b] = pltpu.async_copy(
            bufa[b], out_hbm.at[pl.ds(t0w + c * CT, CT)], sem2)
        if c + 2 < nch:
            w[b].wait()
            g[b] = startc(c + 2)
    for h in w:
        if h is not None:
            h.wait()


def kernel(x, Wr, gate, up, down):
    b, s, d = x.shape
    xf = x.reshape(T, D)

    bt_r = 1024
    logits, i12, w12 = pl.pallas_call(
        _router_body,
        grid=(T // bt_r,),
        in_specs=[
            pl.BlockSpec((bt_r, D), lambda i: (i, 0)),
            pl.BlockSpec((E, D), lambda i: (0, 0)),
        ],
        out_specs=[
            pl.BlockSpec((bt_r, E), lambda i: (i, 0)),
            pl.BlockSpec((bt_r, KTOP), lambda i: (i, 0)),
            pl.BlockSpec((bt_r, KTOP), lambda i: (i, 0)),
        ],
        out_shape=[
            jax.ShapeDtypeStruct((T, E), jnp.float32),
            jax.ShapeDtypeStruct((T, KTOP), jnp.int32),
            jax.ShapeDtypeStruct((T, KTOP), jnp.float32),
        ],
    )(xf, Wr)

    ti_sq = i12.T.reshape(SQ, SQ)
    tw_flat = w12.T.reshape(A)

    dest_sq, eid_pad, nlive_sq = pl.pallas_call(
        _dispatch_body,
        grid=(1,),
        in_specs=[pl.BlockSpec((SQ, SQ), lambda i: (0, 0))],
        out_specs=[
            pl.BlockSpec((SQ, SQ), lambda i: (0, 0)),
            pl.BlockSpec((1, SQ), lambda i: (0, 0)),
            pl.BlockSpec((1, 1), lambda i: (0, 0)),
        ],
        out_shape=[
            jax.ShapeDtypeStruct((SQ, SQ), jnp.int32),
            jax.ShapeDtypeStruct((1, SQ), jnp.int32),
            jax.ShapeDtypeStruct((1, 1), jnp.int32),
        ],
    )(ti_sq)
    dest_flat = dest_sq.reshape(A)
    tile_eid = eid_pad.reshape(SQ)[:NL]

    sc_mesh = plsc.VectorSubcoreMesh(core_axis_name="c", subcore_axis_name="s", num_cores=NC, num_subcores=NS)
    xs, ws = pl.kernel(
        _sc_dispatch_body,
        out_type=[
            jax.ShapeDtypeStruct((L, D), jnp.float32),
            jax.ShapeDtypeStruct((L,), jnp.float32),
        ],
        mesh=sc_mesh,
        scratch_types=[
            pltpu.VMEM((L // NS,), jnp.int32),
            pltpu.VMEM((L // NS,), jnp.float32),
            pltpu.VMEM((SQ // NS, SQ), jnp.int32),
            pltpu.VMEM((SQ // NS * SQ,), jnp.float32),
            pltpu.VMEM((SQ // NS * SQ,), jnp.int32),
            pltpu.VMEM((RPW,), jnp.int32),
            pltpu.VMEM((GC, D), jnp.float32),
            pltpu.VMEM((GC, D), jnp.float32),
            pltpu.VMEM((GC, D), jnp.float32),
            pltpu.VMEM((RPW,), jnp.float32),
            pltpu.VMEM_SHARED((L,), jnp.int32),
            pltpu.VMEM_SHARED((L,), jnp.float32),
            pltpu.SemaphoreType.DMA,
            pltpu.SemaphoreType.DMA,
        ],
    )(dest_sq, tw_flat, xf)

    gate_b = gate.astype(jnp.bfloat16)
    up_b = up.astype(jnp.bfloat16)
    down_b = down.astype(jnp.bfloat16)
    ws2 = ws.reshape(L, 1)

    grid_spec = pltpu.PrefetchScalarGridSpec(
        num_scalar_prefetch=2,
        grid=(NL,),
        in_specs=[
            pl.BlockSpec((BT, D), lambda i, eid, nl: (i, 0)),
            pl.BlockSpec((BT, 1), lambda i, eid, nl: (i, 0)),
            pl.BlockSpec((1, FF, D), lambda i, eid, nl: (eid[i], 0, 0)),
            pl.BlockSpec((1, FF, D), lambda i, eid, nl: (eid[i], 0, 0)),
            pl.BlockSpec((1, D, FF), lambda i, eid, nl: (eid[i], 0, 0)),
        ],
        out_specs=pl.BlockSpec((BT, D), lambda i, eid, nl: (i, 0)),
    )
    ys = pl.pallas_call(
        _moe_ffn_body,
        grid_spec=grid_spec,
        out_shape=jax.ShapeDtypeStruct((L, D), jnp.float32),
        compiler_params=pltpu.CompilerParams(
            dimension_semantics=("arbitrary",)),
    )(tile_eid, nlive_sq.reshape(1), xs, ws2, gate_b, up_b, down_b)

    final = pl.kernel(
        _sc_combine_body,
        out_type=jax.ShapeDtypeStruct((T, D), jnp.float32),
        mesh=sc_mesh,
        scratch_types=[
            pltpu.VMEM((TPW,), jnp.int32),
            pltpu.VMEM((TPW,), jnp.int32),
            pltpu.VMEM((CT, D), jnp.float32),
            pltpu.VMEM((CT, D), jnp.float32),
            pltpu.VMEM((CT, D), jnp.float32),
            pltpu.VMEM((CT, D), jnp.float32),
            pltpu.SemaphoreType.DMA,
            pltpu.SemaphoreType.DMA,
        ],
    )(ys, dest_flat)

    return final.reshape(b, s, d), logits
